# baseline (speedup 1.0000x reference)
import functools

import jax
import jax.numpy as jnp
from jax import lax
from jax.experimental import pallas as pl
from jax.experimental.pallas import tpu as pltpu
from jax.experimental.pallas import tpu_sc as plsc

N = 320000
D = 128
NUM_LABELS = 10000


TC_BLK = 32000
assert N % TC_BLK == 0


def _tc_body(x_ref, w_ref, b_ref, s_ref, out_ref):
    i = pl.program_id(0)

    @pl.when(i == 0)
    def _():
        out_ref[pl.ds(N, N_PAD - N)] = jnp.zeros((N_PAD - N,), jnp.float32)

    h = jnp.dot(x_ref[...], w_ref[...], preferred_element_type=jnp.float32)
    r = jnp.maximum(h + b_ref[...], 0.0)
    per = lax.dot_general(s_ref[...], r, (((1,), (1,)), ((), ())),
                          preferred_element_type=jnp.float32)
    out_ref[pl.ds(i * TC_BLK, TC_BLK)] = per.reshape(TC_BLK)


def _tc_per(inputs, W_emb, b_emb, w_red):
    grid = (N // TC_BLK,)
    aw = jnp.abs(w_red)
    w_scaled = W_emb * aw[None, :]
    b_scaled = (b_emb * aw).reshape(1, D)
    sgn = jnp.where(w_red < 0, -1.0, 1.0).reshape(1, D)
    return pl.pallas_call(
        _tc_body,
        grid=grid,
        in_specs=[
            pl.BlockSpec((TC_BLK, D), lambda i: (i, 0)),
            pl.BlockSpec((D, D), lambda i: (0, 0)),
            pl.BlockSpec((1, D), lambda i: (0, 0)),
            pl.BlockSpec((1, D), lambda i: (0, 0)),
        ],
        out_specs=pl.BlockSpec((N_PAD,), lambda i: (0,)),
        out_shape=jax.ShapeDtypeStruct((N_PAD,), jnp.float32),
    )(inputs, w_scaled, b_scaled, sgn)


NC, NS, L = 2, 16, 16
ROW = 128
NROWS = N // ROW
ROWS_PER_TILE = 160
ROWS_PAD = NS * ROWS_PER_TILE
N_PAD = ROWS_PAD * ROW
NL_PAD = 10240
OUT_CHUNK = NL_PAD // (NC * NS)
ZCHUNK = NL_PAD // NS
SC_GRP = 10


def _sc_segment_sum(vals2d, labs2d):
    mesh = plsc.VectorSubcoreMesh(core_axis_name="c", subcore_axis_name="s")

    @functools.partial(
        pl.kernel,
        out_type=jax.ShapeDtypeStruct((NL_PAD,), jnp.float32),
        mesh=mesh,
        scratch_types=[
            pltpu.VMEM((ROWS_PER_TILE, ROW), jnp.float32),
            pltpu.VMEM((ROWS_PER_TILE, ROW), jnp.int32),
            pltpu.VMEM((ZCHUNK,), jnp.float32),
            pltpu.VMEM_SHARED((NL_PAD,), jnp.float32),
            pltpu.SemaphoreType.DMA,
        ],
    )
    def seg_kernel(vals_hbm, labs_hbm, out_hbm, vals_v, labs_v,
                   zero_v, acc_sh, sem):
        c = lax.axis_index("c")
        s = lax.axis_index("s")

        base = s * ROWS_PER_TILE
        pltpu.sync_copy(vals_hbm.at[pl.ds(base, ROWS_PER_TILE)], vals_v)
        pltpu.sync_copy(labs_hbm.at[pl.ds(base, ROWS_PER_TILE)], labs_v)

        def zbody(i, carry):
            zero_v[pl.ds(i * L, L)] = jnp.zeros((L,), jnp.float32)
            return carry

        lax.fori_loop(0, ZCHUNK // L, zbody, 0)
        pltpu.sync_copy(zero_v, acc_sh.at[pl.ds(s * ZCHUNK, ZCHUNK)])
        plsc.subcore_barrier()

        def sgroup(g, carry):
            copies = []
            for j in range(SC_GRP):
                r = g * SC_GRP + j
                copies.append(pltpu.async_copy(
                    vals_v.at[r], acc_sh.at[labs_v.at[r]], sem, add=True))
            for cp in copies:
                cp.wait()
            return carry

        lax.fori_loop(0, ROWS_PER_TILE // SC_GRP, sgroup, 0)
        plsc.subcore_barrier()

        off = c * (NL_PAD // NC) + s * OUT_CHUNK
        pltpu.sync_copy(acc_sh.at[pl.ds(off, OUT_CHUNK)],
                        zero_v.at[pl.ds(0, OUT_CHUNK)])
        pltpu.sync_copy(zero_v.at[pl.ds(0, OUT_CHUNK)],
                        out_hbm.at[pl.ds(off, OUT_CHUNK)])

    return seg_kernel(vals2d, labs2d)


def kernel(inputs, labels, W_emb, b_emb, w_red):
    per = _tc_per(inputs, W_emb, b_emb, w_red)
    per2d = per.reshape(ROWS_PAD, ROW)
    labs2d = jnp.pad(labels.astype(jnp.int32).reshape(NROWS, ROW),
                     ((0, ROWS_PAD - NROWS), (0, 0)))
    out = _sc_segment_sum(per2d, labs2d)
    return out[:NUM_LABELS]


# speedup vs baseline: 1.0178x; 1.0178x over previous
import functools

import jax
import jax.numpy as jnp
from jax import lax
from jax.experimental import pallas as pl
from jax.experimental.pallas import tpu as pltpu
from jax.experimental.pallas import tpu_sc as plsc

N = 320000
D = 128
NUM_LABELS = 10000


TC_BLK = 32000
assert N % TC_BLK == 0


def _tc_body(x_ref, w_ref, b_ref, s_ref, out_ref):
    i = pl.program_id(0)
    h = jnp.dot(x_ref[...], w_ref[...], preferred_element_type=jnp.float32)
    r = jnp.maximum(h + b_ref[...], 0.0)
    per = lax.dot_general(s_ref[...], r, (((1,), (1,)), ((), ())),
                          preferred_element_type=jnp.float32)
    out_ref[pl.ds(i * TC_BLK, TC_BLK)] = per.reshape(TC_BLK)


def _tc_per(inputs, W_emb, b_emb, w_red):
    aw = jnp.abs(w_red)
    w_scaled = W_emb * aw[None, :]
    b_scaled = (b_emb * aw).reshape(1, D)
    sgn = jnp.where(w_red < 0, -1.0, 1.0).reshape(1, D)
    return pl.pallas_call(
        _tc_body,
        grid=(N // TC_BLK,),
        in_specs=[
            pl.BlockSpec((TC_BLK, D), lambda i: (i, 0)),
            pl.BlockSpec((D, D), lambda i: (0, 0)),
            pl.BlockSpec((1, D), lambda i: (0, 0)),
            pl.BlockSpec((1, D), lambda i: (0, 0)),
        ],
        out_specs=pl.BlockSpec((N,), lambda i: (0,)),
        out_shape=jax.ShapeDtypeStruct((N,), jnp.float32),
    )(inputs, w_scaled, b_scaled, sgn)


NC, NS, L = 2, 16, 16
ROW = 128
NROWS = N // ROW
ROWS_PER_TILE = 160
MAIN_ROWS = 96
TAIL_ROWS = 4
NL_PAD = 10240
OUT_CHUNK = NL_PAD // (NC * NS)
ZCHUNK = NL_PAD // NS
SC_GRP = 8


def _sc_segment_sum(vals, labs2d, labs_tail):
    mesh = plsc.VectorSubcoreMesh(core_axis_name="c", subcore_axis_name="s")

    @functools.partial(
        pl.kernel,
        out_type=jax.ShapeDtypeStruct((NL_PAD,), jnp.float32),
        mesh=mesh,
        scratch_types=[
            pltpu.VMEM((ROWS_PER_TILE * ROW,), jnp.float32),
            pltpu.VMEM((ROWS_PER_TILE, ROW), jnp.int32),
            pltpu.VMEM((TAIL_ROWS, ROW), jnp.int32),
            pltpu.VMEM((ZCHUNK,), jnp.float32),
            pltpu.VMEM_SHARED((NL_PAD,), jnp.float32),
            pltpu.SemaphoreType.DMA,
        ],
    )
    def seg_kernel(vals_hbm, labs_hbm, tail_hbm, out_hbm, vals_v, labs_v,
                   tail_v, zero_v, acc_sh, sem):
        c = lax.axis_index("c")
        s = lax.axis_index("s")

        base = s * ROWS_PER_TILE

        @pl.when(s < NS - 1)
        def _():
            pltpu.sync_copy(vals_hbm.at[pl.ds(base * ROW,
                                              ROWS_PER_TILE * ROW)], vals_v)
            pltpu.sync_copy(labs_hbm.at[pl.ds(base, ROWS_PER_TILE)], labs_v)

        @pl.when(s == NS - 1)
        def _():
            pltpu.sync_copy(vals_hbm.at[pl.ds(base * ROW, MAIN_ROWS * ROW)],
                            vals_v.at[pl.ds(0, MAIN_ROWS * ROW)])
            pltpu.sync_copy(
                vals_hbm.at[pl.ds((base + MAIN_ROWS) * ROW,
                                  TAIL_ROWS * ROW)],
                vals_v.at[pl.ds(MAIN_ROWS * ROW, TAIL_ROWS * ROW)])
            pltpu.sync_copy(labs_hbm.at[pl.ds(base, MAIN_ROWS)],
                            labs_v.at[pl.ds(0, MAIN_ROWS)])
            pltpu.sync_copy(tail_hbm, tail_v)

        def zbody(i, carry):
            zero_v[pl.ds(i * L, L)] = jnp.zeros((L,), jnp.float32)
            return carry

        lax.fori_loop(0, ZCHUNK // L, zbody, 0)
        pltpu.sync_copy(zero_v, acc_sh.at[pl.ds(s * ZCHUNK, ZCHUNK)])
        plsc.subcore_barrier()

        def sgroup(g, carry):
            copies = []
            for j in range(SC_GRP):
                r = g * SC_GRP + j
                copies.append(pltpu.async_copy(
                    vals_v.at[pl.ds(r * ROW, ROW)], acc_sh.at[labs_v.at[r]],
                    sem, add=True))
            for cp in copies:
                cp.wait()
            return carry

        @pl.when(s < NS - 1)
        def _():
            lax.fori_loop(0, ROWS_PER_TILE // SC_GRP, sgroup, 0)

        @pl.when(s == NS - 1)
        def _():
            lax.fori_loop(0, MAIN_ROWS // SC_GRP, sgroup, 0)
            tail = []
            for j in range(TAIL_ROWS):
                tail.append(pltpu.async_copy(
                    vals_v.at[pl.ds((MAIN_ROWS + j) * ROW, ROW)],
                    acc_sh.at[tail_v.at[j]], sem, add=True))
            for cp in tail:
                cp.wait()

        plsc.subcore_barrier()

        off = c * (NL_PAD // NC) + s * OUT_CHUNK
        pltpu.sync_copy(acc_sh.at[pl.ds(off, OUT_CHUNK)],
                        zero_v.at[pl.ds(0, OUT_CHUNK)])
        pltpu.sync_copy(zero_v.at[pl.ds(0, OUT_CHUNK)],
                        out_hbm.at[pl.ds(off, OUT_CHUNK)])

    return seg_kernel(vals, labs2d, labs_tail)


def kernel(inputs, labels, W_emb, b_emb, w_red):
    per = _tc_per(inputs, W_emb, b_emb, w_red)
    labs2d = labels.astype(jnp.int32).reshape(NROWS, ROW)
    labs_tail = lax.slice(labs2d, (NROWS - TAIL_ROWS, 0), (NROWS, ROW))
    out = _sc_segment_sum(per, labs2d, labs_tail)
    return out[:NUM_LABELS]


# speedup vs baseline: 1.0181x; 1.0003x over previous
import functools

import jax
import jax.numpy as jnp
from jax import lax
from jax.experimental import pallas as pl
from jax.experimental.pallas import tpu as pltpu
from jax.experimental.pallas import tpu_sc as plsc

N = 320000
D = 128
NUM_LABELS = 10000


TC_BLK = 32000
assert N % TC_BLK == 0


def _tc_body(x_ref, w_ref, b_ref, s_ref, out_ref):
    i = pl.program_id(0)
    h = jnp.dot(x_ref[...], w_ref[...], preferred_element_type=jnp.float32)
    r = jnp.maximum(h + b_ref[...], 0.0)
    per = lax.dot_general(s_ref[...], r, (((1,), (1,)), ((), ())),
                          preferred_element_type=jnp.float32)
    out_ref[pl.ds(i * TC_BLK, TC_BLK)] = per.reshape(TC_BLK)


def _tc_per(inputs, W_emb, b_emb, w_red):
    aw = jnp.abs(w_red)
    w_scaled = W_emb * aw[None, :]
    b_scaled = (b_emb * aw).reshape(1, D)
    sgn = jnp.where(w_red < 0, -1.0, 1.0).reshape(1, D)
    return pl.pallas_call(
        _tc_body,
        grid=(N // TC_BLK,),
        in_specs=[
            pl.BlockSpec((TC_BLK, D), lambda i: (i, 0)),
            pl.BlockSpec((D, D), lambda i: (0, 0)),
            pl.BlockSpec((1, D), lambda i: (0, 0)),
            pl.BlockSpec((1, D), lambda i: (0, 0)),
        ],
        out_specs=pl.BlockSpec((N,), lambda i: (0,)),
        out_shape=jax.ShapeDtypeStruct((N,), jnp.float32),
    )(inputs, w_scaled, b_scaled, sgn)


NC, NS, L = 2, 16, 16
ROW = 128
NROWS = N // ROW
ROWS_PER_TILE = 160
MAIN_ROWS = 96
TAIL_ROWS = 4
NL_PAD = 10240
OUT_CHUNK = NL_PAD // (NC * NS)
ZCHUNK = NL_PAD // NS
SC_GRP = 16


def _sc_segment_sum(vals, labs2d, labs_tail):
    mesh = plsc.VectorSubcoreMesh(core_axis_name="c", subcore_axis_name="s")

    @functools.partial(
        pl.kernel,
        out_type=jax.ShapeDtypeStruct((NL_PAD,), jnp.float32),
        mesh=mesh,
        scratch_types=[
            pltpu.VMEM((ROWS_PER_TILE * ROW,), jnp.float32),
            pltpu.VMEM((ROWS_PER_TILE, ROW), jnp.int32),
            pltpu.VMEM((TAIL_ROWS, ROW), jnp.int32),
            pltpu.VMEM((ZCHUNK,), jnp.float32),
            pltpu.VMEM_SHARED((NL_PAD,), jnp.float32),
            pltpu.SemaphoreType.DMA,
        ],
    )
    def seg_kernel(vals_hbm, labs_hbm, tail_hbm, out_hbm, vals_v, labs_v,
                   tail_v, zero_v, acc_sh, sem):
        c = lax.axis_index("c")
        s = lax.axis_index("s")

        base = s * ROWS_PER_TILE

        @pl.when(s < NS - 1)
        def _():
            pltpu.sync_copy(vals_hbm.at[pl.ds(base * ROW,
                                              ROWS_PER_TILE * ROW)], vals_v)
            pltpu.sync_copy(labs_hbm.at[pl.ds(base, ROWS_PER_TILE)], labs_v)

        @pl.when(s == NS - 1)
        def _():
            pltpu.sync_copy(vals_hbm.at[pl.ds(base * ROW, MAIN_ROWS * ROW)],
                            vals_v.at[pl.ds(0, MAIN_ROWS * ROW)])
            pltpu.sync_copy(
                vals_hbm.at[pl.ds((base + MAIN_ROWS) * ROW,
                                  TAIL_ROWS * ROW)],
                vals_v.at[pl.ds(MAIN_ROWS * ROW, TAIL_ROWS * ROW)])
            pltpu.sync_copy(labs_hbm.at[pl.ds(base, MAIN_ROWS)],
                            labs_v.at[pl.ds(0, MAIN_ROWS)])
            pltpu.sync_copy(tail_hbm, tail_v)

        def zbody(i, carry):
            zero_v[pl.ds(i * L, L)] = jnp.zeros((L,), jnp.float32)
            return carry

        lax.fori_loop(0, ZCHUNK // L, zbody, 0)
        pltpu.sync_copy(zero_v, acc_sh.at[pl.ds(s * ZCHUNK, ZCHUNK)])
        plsc.subcore_barrier()

        def sgroup(g, carry):
            copies = []
            for j in range(SC_GRP):
                r = g * SC_GRP + j
                copies.append(pltpu.async_copy(
                    vals_v.at[pl.ds(r * ROW, ROW)], acc_sh.at[labs_v.at[r]],
                    sem, add=True))
            for cp in copies:
                cp.wait()
            return carry

        @pl.when(s < NS - 1)
        def _():
            lax.fori_loop(0, ROWS_PER_TILE // SC_GRP, sgroup, 0)

        @pl.when(s == NS - 1)
        def _():
            lax.fori_loop(0, MAIN_ROWS // SC_GRP, sgroup, 0)
            tail = []
            for j in range(TAIL_ROWS):
                tail.append(pltpu.async_copy(
                    vals_v.at[pl.ds((MAIN_ROWS + j) * ROW, ROW)],
                    acc_sh.at[tail_v.at[j]], sem, add=True))
            for cp in tail:
                cp.wait()

        plsc.subcore_barrier()

        off = c * (NL_PAD // NC) + s * OUT_CHUNK
        pltpu.sync_copy(acc_sh.at[pl.ds(off, OUT_CHUNK)],
                        zero_v.at[pl.ds(0, OUT_CHUNK)])
        pltpu.sync_copy(zero_v.at[pl.ds(0, OUT_CHUNK)],
                        out_hbm.at[pl.ds(off, OUT_CHUNK)])

    return seg_kernel(vals, labs2d, labs_tail)


def kernel(inputs, labels, W_emb, b_emb, w_red):
    per = _tc_per(inputs, W_emb, b_emb, w_red)
    labs2d = labels.astype(jnp.int32).reshape(NROWS, ROW)
    labs_tail = lax.slice(labs2d, (NROWS - TAIL_ROWS, 0), (NROWS, ROW))
    out = _sc_segment_sum(per, labs2d, labs_tail)
    return out[:NUM_LABELS]


# speedup vs baseline: 1.0369x; 1.0185x over previous
import functools

import jax
import jax.numpy as jnp
from jax import lax
from jax.experimental import pallas as pl
from jax.experimental.pallas import tpu as pltpu
from jax.experimental.pallas import tpu_sc as plsc

N = 320000
D = 128
NUM_LABELS = 10000


TC_BLK = 16000
TC_STEPS = N // TC_BLK
NBUF = 3
assert N % TC_BLK == 0


def _tc_body(x_hbm, w_ref, b_ref, s_ref, out_ref, xbuf, sems):
    i = pl.program_id(0)

    def start(blk, slot):
        pltpu.make_async_copy(
            x_hbm.at[pl.ds(blk * TC_BLK, TC_BLK), :],
            xbuf.at[slot], sems.at[slot]).start()

    @pl.when(i == 0)
    def _():
        for k in range(NBUF):
            start(k, k)

    slot = lax.rem(i, NBUF)
    pltpu.make_async_copy(
        x_hbm.at[pl.ds(i * TC_BLK, TC_BLK), :],
        xbuf.at[slot], sems.at[slot]).wait()

    h = jnp.dot(xbuf[slot], w_ref[...], preferred_element_type=jnp.float32)
    r = jnp.maximum(h + b_ref[...], 0.0)
    per = lax.dot_general(s_ref[...], r, (((1,), (1,)), ((), ())),
                          preferred_element_type=jnp.float32)
    out_ref[pl.ds(i * TC_BLK, TC_BLK)] = per.reshape(TC_BLK)

    @pl.when(i + NBUF < TC_STEPS)
    def _():
        start(i + NBUF, slot)


def _tc_per(inputs, W_emb, b_emb, w_red):
    aw = jnp.abs(w_red)
    w_scaled = W_emb * aw[None, :]
    b_scaled = (b_emb * aw).reshape(1, D)
    sgn = jnp.where(w_red < 0, -1.0, 1.0).reshape(1, D)
    return pl.pallas_call(
        _tc_body,
        grid=(TC_STEPS,),
        in_specs=[
            pl.BlockSpec(memory_space=pl.ANY),
            pl.BlockSpec((D, D), lambda i: (0, 0)),
            pl.BlockSpec((1, D), lambda i: (0, 0)),
            pl.BlockSpec((1, D), lambda i: (0, 0)),
        ],
        out_specs=pl.BlockSpec((N,), lambda i: (0,)),
        out_shape=jax.ShapeDtypeStruct((N,), jnp.float32),
        scratch_shapes=[
            pltpu.VMEM((NBUF, TC_BLK, D), jnp.float32),
            pltpu.SemaphoreType.DMA((NBUF,)),
        ],
    )(inputs, w_scaled, b_scaled, sgn)


NC, NS, L = 2, 16, 16
ROW = 128
NROWS = N // ROW
ROWS_PER_TILE = 160
MAIN_ROWS = 96
TAIL_ROWS = 4
NL_PAD = 10240
OUT_CHUNK = NL_PAD // (NC * NS)
ZCHUNK = NL_PAD // NS
SC_GRP = 16


def _sc_segment_sum(vals, labs2d, labs_tail):
    mesh = plsc.VectorSubcoreMesh(core_axis_name="c", subcore_axis_name="s")

    @functools.partial(
        pl.kernel,
        out_type=jax.ShapeDtypeStruct((NL_PAD,), jnp.float32),
        mesh=mesh,
        scratch_types=[
            pltpu.VMEM((ROWS_PER_TILE * ROW,), jnp.float32),
            pltpu.VMEM((ROWS_PER_TILE, ROW), jnp.int32),
            pltpu.VMEM((TAIL_ROWS, ROW), jnp.int32),
            pltpu.VMEM((ZCHUNK,), jnp.float32),
            pltpu.VMEM_SHARED((NL_PAD,), jnp.float32),
            pltpu.SemaphoreType.DMA,
        ],
    )
    def seg_kernel(vals_hbm, labs_hbm, tail_hbm, out_hbm, vals_v, labs_v,
                   tail_v, zero_v, acc_sh, sem):
        c = lax.axis_index("c")
        s = lax.axis_index("s")

        base = s * ROWS_PER_TILE

        @pl.when(s < NS - 1)
        def _():
            pltpu.sync_copy(vals_hbm.at[pl.ds(base * ROW,
                                              ROWS_PER_TILE * ROW)], vals_v)
            pltpu.sync_copy(labs_hbm.at[pl.ds(base, ROWS_PER_TILE)], labs_v)

        @pl.when(s == NS - 1)
        def _():
            pltpu.sync_copy(vals_hbm.at[pl.ds(base * ROW, MAIN_ROWS * ROW)],
                            vals_v.at[pl.ds(0, MAIN_ROWS * ROW)])
            pltpu.sync_copy(
                vals_hbm.at[pl.ds((base + MAIN_ROWS) * ROW,
                                  TAIL_ROWS * ROW)],
                vals_v.at[pl.ds(MAIN_ROWS * ROW, TAIL_ROWS * ROW)])
            pltpu.sync_copy(labs_hbm.at[pl.ds(base, MAIN_ROWS)],
                            labs_v.at[pl.ds(0, MAIN_ROWS)])
            pltpu.sync_copy(tail_hbm, tail_v)

        def zbody(i, carry):
            zero_v[pl.ds(i * L, L)] = jnp.zeros((L,), jnp.float32)
            return carry

        lax.fori_loop(0, ZCHUNK // L, zbody, 0)
        pltpu.sync_copy(zero_v, acc_sh.at[pl.ds(s * ZCHUNK, ZCHUNK)])
        plsc.subcore_barrier()

        def sgroup(g, carry):
            copies = []
            for j in range(SC_GRP):
                r = g * SC_GRP + j
                copies.append(pltpu.async_copy(
                    vals_v.at[pl.ds(r * ROW, ROW)], acc_sh.at[labs_v.at[r]],
                    sem, add=True))
            for cp in copies:
                cp.wait()
            return carry

        @pl.when(s < NS - 1)
        def _():
            lax.fori_loop(0, ROWS_PER_TILE // SC_GRP, sgroup, 0)

        @pl.when(s == NS - 1)
        def _():
            lax.fori_loop(0, MAIN_ROWS // SC_GRP, sgroup, 0)
            tail = []
            for j in range(TAIL_ROWS):
                tail.append(pltpu.async_copy(
                    vals_v.at[pl.ds((MAIN_ROWS + j) * ROW, ROW)],
                    acc_sh.at[tail_v.at[j]], sem, add=True))
            for cp in tail:
                cp.wait()

        plsc.subcore_barrier()

        off = c * (NL_PAD // NC) + s * OUT_CHUNK
        pltpu.sync_copy(acc_sh.at[pl.ds(off, OUT_CHUNK)],
                        zero_v.at[pl.ds(0, OUT_CHUNK)])
        pltpu.sync_copy(zero_v.at[pl.ds(0, OUT_CHUNK)],
                        out_hbm.at[pl.ds(off, OUT_CHUNK)])

    return seg_kernel(vals, labs2d, labs_tail)


def kernel(inputs, labels, W_emb, b_emb, w_red):
    per = _tc_per(inputs, W_emb, b_emb, w_red)
    labs2d = labels.astype(jnp.int32).reshape(NROWS, ROW)
    labs_tail = lax.slice(labs2d, (NROWS - TAIL_ROWS, 0), (NROWS, ROW))
    out = _sc_segment_sum(per, labs2d, labs_tail)
    return out[:NUM_LABELS]
